# baseline (speedup 1.0000x reference)
import jax
import jax.numpy as jnp
from jax.experimental import pallas as pl
from jax.experimental.pallas import tpu as pltpu
from jax.experimental.pallas import tpu_sc as plsc

_BBLK = 8
_B_SC = 3072


def _sc_gather(ids, matrix):
    b, s = ids.shape
    n, d = matrix.shape
    nblocks = b // _BBLK
    indices = ids.reshape(nblocks, _BBLK, s)

    mesh = plsc.VectorSubcoreMesh(
        core_axis_name="core", subcore_axis_name="subcore"
    )

    @pl.kernel(
        out_type=jax.ShapeDtypeStruct((b, s, d), matrix.dtype),
        mesh=mesh,
    )
    def gather_kernel(x_hbm, i_hbm, o_hbm):
        def body(i_vmem, o_vmem):
            @pl.loop(0, _BBLK)
            def _(j):
                pltpu.sync_copy(x_hbm.at[i_vmem.at[0, j]], o_vmem.at[j])

        pltpu.emit_pipeline(
            body,
            grid=(nblocks,),
            in_specs=[
                pl.BlockSpec((1, _BBLK, s), index_map=lambda i: (i, 0, 0))
            ],
            out_specs=[
                pl.BlockSpec((_BBLK, s, d), index_map=lambda i: (i, 0, 0))
            ],
            core_axis_name=("core", "subcore"),
            dimension_semantics=(pltpu.PARALLEL,),
            trace_scopes=False,
        )(i_hbm, o_hbm)

    return gather_kernel(matrix, indices)


def _tc_gather(ids, matrix):
    b, s = ids.shape
    n, d = matrix.shape
    nblocks = b // _BBLK
    indices = ids.reshape(nblocks, _BBLK, s)

    def body(i_ref, x_ref, o_ref):
        def batch_row(i, _):
            def tok(j, _):
                o_ref[i, j] = x_ref[i_ref[0, i, j]]
                return 0

            return jax.lax.fori_loop(0, s, tok, 0)

        jax.lax.fori_loop(0, _BBLK, batch_row, 0)

    return pl.pallas_call(
        body,
        grid=(nblocks,),
        in_specs=[
            pl.BlockSpec(
                (1, _BBLK, s),
                index_map=lambda i: (i, 0, 0),
                memory_space=pltpu.SMEM,
            ),
            pl.BlockSpec((n, d), index_map=lambda i: (0, 0)),
        ],
        out_specs=pl.BlockSpec((_BBLK, s, d), index_map=lambda i: (i, 0, 0)),
        out_shape=jax.ShapeDtypeStruct((b, s, d), matrix.dtype),
    )(indices, matrix)


def kernel(token_ids, matrix):
    ids = token_ids.astype(jnp.int32)
    sc_out = _sc_gather(ids[:_B_SC], matrix)
    tc_out = _tc_gather(ids[_B_SC:], matrix)
    return jnp.concatenate([sc_out, tc_out], axis=0)


# speedup vs baseline: 3.8556x; 3.8556x over previous
import jax
import jax.numpy as jnp
from jax.experimental import pallas as pl
from jax.experimental.pallas import tpu as pltpu
from jax.experimental.pallas import tpu_sc as plsc

_BBLK = 8


def kernel(token_ids, matrix):
    b, s = token_ids.shape
    n, d = matrix.shape
    nblocks = b // _BBLK
    indices = token_ids.astype(jnp.int32).reshape(nblocks, _BBLK, s)

    mesh = plsc.VectorSubcoreMesh(
        core_axis_name="core", subcore_axis_name="subcore"
    )

    @pl.kernel(
        out_type=jax.ShapeDtypeStruct((b, s, d), matrix.dtype),
        mesh=mesh,
        scratch_types=[pltpu.SemaphoreType.DMA],
    )
    def gather_kernel(x_hbm, i_hbm, o_hbm, gsem):
        def body(i_vmem, o_vmem):
            copies = [
                pltpu.async_copy(
                    x_hbm.at[i_vmem.at[0, j]], o_vmem.at[j], gsem
                )
                for j in range(_BBLK)
            ]
            for c in copies:
                c.wait()

        pltpu.emit_pipeline(
            body,
            grid=(nblocks,),
            in_specs=[
                pl.BlockSpec((1, _BBLK, s), index_map=lambda i: (i, 0, 0))
            ],
            out_specs=[
                pl.BlockSpec((_BBLK, s, d), index_map=lambda i: (i, 0, 0))
            ],
            core_axis_name=("core", "subcore"),
            dimension_semantics=(pltpu.PARALLEL,),
            trace_scopes=False,
        )(i_hbm, o_hbm)

    return gather_kernel(matrix, indices)
